# Initial kernel scaffold; baseline (speedup 1.0000x reference)
import jax
import jax.numpy as jnp
from jax.experimental import pallas as pl


def kernel(x0, x1, x2, adj0_idx, adj1_idx, adj2_idx, inc1_idx, inc2_idx, W_same_0, W_same_1, W_same_2, W_h2l_0, W_h2l_1, W_l2h_1, W_l2h_2):
    raise NotImplementedError("write your pallas kernel here")


# speedup vs baseline: 1.5674x; 1.5674x over previous
import jax
import jax.numpy as jnp
from jax import lax
from jax.experimental import pallas as pl
from jax.experimental.pallas import tpu as pltpu
from jax.experimental.pallas import tpu_sc as plsc

N0, N1, N2 = 10000, 30000, 20000
C = 128
NC, NS = 2, 16
W = 64
DUMP = NS
ACC_ROWS = N1 // NC + DUMP
FCH = 8


def _mm_body(x_ref, *refs):
    k = len(refs) // 2
    x = x_ref[...]
    for w_ref, o_ref in zip(refs[:k], refs[k:]):
        o_ref[...] = jnp.dot(x, w_ref[...], preferred_element_type=jnp.float32)


def _matmuls(x, ws, block=1000):
    n = x.shape[0]
    k = len(ws)
    return pl.pallas_call(
        _mm_body,
        grid=(n // block,),
        in_specs=[pl.BlockSpec((block, C), lambda i: (i, 0))]
        + [pl.BlockSpec((C, C), lambda i: (0, 0))] * k,
        out_specs=[pl.BlockSpec((block, C), lambda i: (i, 0))] * k,
        out_shape=[jax.ShapeDtypeStruct((n, C), jnp.float32)] * k,
    )(x, *ws)


def _sig_body(x_ref, o_ref):
    o_ref[...] = jax.nn.sigmoid(x_ref[...])


def _sigmoid(a, block=1000):
    n = a.shape[0]
    return pl.pallas_call(
        _sig_body,
        grid=(n // block,),
        in_specs=[pl.BlockSpec((block, C), lambda i: (i, 0))],
        out_specs=pl.BlockSpec((block, C), lambda i: (i, 0)),
        out_shape=jax.ShapeDtypeStruct((n, C), jnp.float32),
    )(a)


def _pad_idx(dst, src, n_dst, n_src):
    nnz = dst.shape[0]
    m = (-nnz) % (NS * W)
    dst = jnp.concatenate([dst, jnp.full((m,), n_dst, jnp.int32)])
    src = jnp.concatenate(
        [src, (jnp.arange(m, dtype=jnp.int32) * 997) % n_src])
    return dst, src


_PHASES = (
    (N0, [(0, 0), (1, 1)], 0),
    (N1, [(2, 2), (3, 3), (4, 4)], 1),
    (N2, [(5, 5), (6, 6)], 2),
)


def _sc_body(*args):
    hs = args[0:7]
    idxs = args[7:21]
    outs = args[21:24]
    acc, dst_w, src_w, gbuf, zbuf, sem = args[24:30]

    cid = lax.axis_index("c")
    sid = lax.axis_index("s")

    zv = jnp.zeros((16,), jnp.float32)
    for r in range(FCH):
        for j in range(C // 16):
            zbuf[r, pl.ds(j * 16, 16)] = zv

    for n_out, pass_list, out_i in _PHASES:
        half = n_out // NC
        base = cid * half
        out = outs[out_i]

        nchunks = (half + DUMP) // FCH
        nz_iter = (nchunks + NS - 1) // NS

        def zero_body(i, _, nchunks=nchunks):
            chunk = i * NS + sid

            @pl.when(chunk < nchunks)
            def _():
                pltpu.sync_copy(zbuf, acc.at[pl.ds(chunk * FCH, FCH)])
            return 0

        lax.fori_loop(0, nz_iter, zero_body, 0)
        plsc.subcore_barrier()

        for h_i, idx_i in pass_list:
            h = hs[h_i]
            dst_hbm = idxs[2 * idx_i]
            src_hbm = idxs[2 * idx_i + 1]
            nwin_t = dst_hbm.shape[0] // (W * NS)
            win0 = sid * nwin_t

            def win_body(i, _, h=h, dst_hbm=dst_hbm, src_hbm=src_hbm,
                         win0=win0, base=base, half=half):
                off = (win0 + i) * W
                pltpu.sync_copy(dst_hbm.at[pl.ds(off, W)], dst_w)
                pltpu.sync_copy(src_hbm.at[pl.ds(off, W)], src_w)
                cp = pltpu.async_copy(h.at[src_w], gbuf, sem)
                dump = jnp.full((16,), half, jnp.int32) + sid
                for j in range(W // 16):
                    d = dst_w[pl.ds(j * 16, 16)]
                    l = d - base
                    oor = (l < 0) | (l >= half)
                    dst_w[pl.ds(j * 16, 16)] = jnp.where(oor, dump, l)
                cp.wait()
                pltpu.sync_copy(gbuf, acc.at[dst_w], add=True)
                return 0

            lax.fori_loop(0, nwin_t, win_body, 0)

        plsc.subcore_barrier()

        nf = half // FCH
        nf_iter = (nf + NS - 1) // NS

        def flush_body(i, _, nf=nf, base=base, out=out):
            chunk = i * NS + sid

            @pl.when(chunk < nf)
            def _():
                pltpu.sync_copy(
                    acc.at[pl.ds(chunk * FCH, FCH)],
                    out.at[pl.ds(base + chunk * FCH, FCH)])
            return 0

        lax.fori_loop(0, nf_iter, flush_body, 0)
        plsc.subcore_barrier()


def _sc_scatter(hs, idx_pairs):
    mesh = plsc.VectorSubcoreMesh(core_axis_name="c", subcore_axis_name="s",
                                  num_cores=NC, num_subcores=NS)
    flat_idx = [a for pair in idx_pairs for a in pair]
    f = pl.kernel(
        _sc_body,
        out_type=[
            jax.ShapeDtypeStruct((N0, C), jnp.float32),
            jax.ShapeDtypeStruct((N1, C), jnp.float32),
            jax.ShapeDtypeStruct((N2, C), jnp.float32),
        ],
        mesh=mesh,
        scratch_types=[
            pltpu.VMEM_SHARED((ACC_ROWS, C), jnp.float32),
            pltpu.VMEM((W,), jnp.int32),
            pltpu.VMEM((W,), jnp.int32),
            pltpu.VMEM((W, C), jnp.float32),
            pltpu.VMEM((FCH, C), jnp.float32),
            pltpu.SemaphoreType.DMA,
        ],
    )
    return f(*hs, *flat_idx)


@jax.jit
def kernel(x0, x1, x2, adj0_idx, adj1_idx, adj2_idx, inc1_idx, inc2_idx,
           W_same_0, W_same_1, W_same_2, W_h2l_0, W_h2l_1, W_l2h_1, W_l2h_2):
    h_s0, h_l2h1 = _matmuls(x0, [W_same_0, W_l2h_1])
    h_s1, h_h2l0, h_l2h2 = _matmuls(x1, [W_same_1, W_h2l_0, W_l2h_2])
    h_s2, h_h2l1 = _matmuls(x2, [W_same_2, W_h2l_1])

    idx_pairs = [
        _pad_idx(adj0_idx[0], adj0_idx[1], N0, N0),
        _pad_idx(inc1_idx[0], inc1_idx[1], N0, N1),
        _pad_idx(adj1_idx[0], adj1_idx[1], N1, N1),
        _pad_idx(inc2_idx[0], inc2_idx[1], N1, N2),
        _pad_idx(inc1_idx[1], inc1_idx[0], N1, N0),
        _pad_idx(adj2_idx[0], adj2_idx[1], N2, N2),
        _pad_idx(inc2_idx[1], inc2_idx[0], N2, N1),
    ]
    hs = (h_s0, h_h2l0, h_s1, h_h2l1, h_l2h1, h_s2, h_l2h2)
    a0, a1, a2 = _sc_scatter(hs, idx_pairs)
    return (_sigmoid(a0), _sigmoid(a1), _sigmoid(a2))


# speedup vs baseline: 2.8317x; 1.8066x over previous
import jax
import jax.numpy as jnp
from jax import lax
from jax.experimental import pallas as pl
from jax.experimental.pallas import tpu as pltpu
from jax.experimental.pallas import tpu_sc as plsc

N0, N1, N2 = 10000, 30000, 20000
C = 128
NC, NS = 2, 16
NW = NC * NS
W = 128
PADM = 2 * W * NW
R = 10000
DUMP = 8
ACC_ROWS = R + DUMP
FCH = 8


def _mm_body(x_ref, *refs):
    k = len(refs) // 2
    x = x_ref[...]
    for w_ref, o_ref in zip(refs[:k], refs[k:]):
        o_ref[...] = jnp.dot(x, w_ref[...], preferred_element_type=jnp.float32)


def _matmuls(x, ws, block=1000):
    n = x.shape[0]
    k = len(ws)
    return pl.pallas_call(
        _mm_body,
        grid=(n // block,),
        in_specs=[pl.BlockSpec((block, C), lambda i: (i, 0))]
        + [pl.BlockSpec((C, C), lambda i: (0, 0))] * k,
        out_specs=[pl.BlockSpec((block, C), lambda i: (i, 0))] * k,
        out_shape=[jax.ShapeDtypeStruct((n, C), jnp.float32)] * k,
    )(x, *ws)


def _merge_body(p_ref, o_ref):
    o_ref[...] = jax.nn.sigmoid(p_ref[0] + p_ref[1])


def _merge_sigmoid(p, block=1000):
    n = p.shape[1]
    return pl.pallas_call(
        _merge_body,
        grid=(n // block,),
        in_specs=[pl.BlockSpec((2, block, C), lambda i: (0, i, 0))],
        out_specs=pl.BlockSpec((block, C), lambda i: (i, 0)),
        out_shape=jax.ShapeDtypeStruct((n, C), jnp.float32),
    )(p)


def _pad_idx(dst, src, n_dst, n_src):
    nnz = dst.shape[0]
    m = (-nnz) % PADM
    dst = jnp.concatenate([dst, jnp.full((m,), n_dst, jnp.int32)])
    src = jnp.concatenate(
        [src, (jnp.arange(m, dtype=jnp.int32) * 997) % n_src])
    return dst, src


_TASKS = (
    (0, 0, [(0, 0), (1, 1)]),
    (1, 0, [(2, 2), (3, 3), (4, 4)]),
    (1, R, [(2, 2), (3, 3), (4, 4)]),
    (1, 2 * R, [(2, 2), (3, 3), (4, 4)]),
    (2, 0, [(5, 5), (6, 6)]),
    (2, R, [(5, 5), (6, 6)]),
)


def _sc_body(*args):
    hs = args[0:7]
    idxs = args[7:21]
    outs = args[21:24]
    (acc, d0, s0, d1, s1, g0, g1, zbuf, sem0, sem1) = args[24:34]

    cid = lax.axis_index("c")
    sid = lax.axis_index("s")
    wid = cid * NS + sid
    dump = jnp.full((16,), R, jnp.int32) + (sid % DUMP)

    zv = jnp.zeros((16,), jnp.float32)
    for r in range(FCH):
        for j in range(C // 16):
            zbuf[r, pl.ds(j * 16, 16)] = zv

    def load(dst_hbm, src_hbm, off, d, s, base):
        pltpu.sync_copy(dst_hbm.at[pl.ds(off, W)], d)
        pltpu.sync_copy(src_hbm.at[pl.ds(off, W)], s)

        def tbody(j, _):
            dd = d[pl.ds(j * 16, 16)]
            l = dd - base
            oor = (l < 0) | (l >= R)
            d[pl.ds(j * 16, 16)] = jnp.where(oor, dump, l)
            return 0

        lax.fori_loop(0, W // 16, tbody, 0)

    for out_i, tbase, pass_list in _TASKS:
        out = outs[out_i]

        nchunks = ACC_ROWS // FCH
        nz_iter = (nchunks + NS - 1) // NS

        def zero_body(i, _, nchunks=nchunks):
            chunk = i * NS + sid

            @pl.when(chunk < nchunks)
            def _():
                pltpu.sync_copy(zbuf, acc.at[pl.ds(chunk * FCH, FCH)])
            return 0

        lax.fori_loop(0, nz_iter, zero_body, 0)
        plsc.subcore_barrier()

        for h_i, idx_i in pass_list:
            h = hs[h_i]
            dst_hbm = idxs[2 * idx_i]
            src_hbm = idxs[2 * idx_i + 1]
            nwt = dst_hbm.shape[0] // (W * NW)
            w0 = wid * nwt

            def ld(i, d, s, h=h, dst_hbm=dst_hbm, src_hbm=src_hbm,
                   w0=w0, tbase=tbase):
                load(dst_hbm, src_hbm, (w0 + i) * W, d, s, tbase)

            def start(s, g, sem, h=h):
                return pltpu.async_copy(h.at[s], g, sem)

            def wait(s, g, sem, h=h):
                pltpu.make_async_copy(h.at[s], g, sem).wait()

            def scat(g, d):
                pltpu.sync_copy(g, acc.at[d], add=True)

            ld(0, d0, s0)
            start(s0, g0, sem0)

            def pair_body(p, _, ld=ld):
                ld(2 * p + 1, d1, s1)
                start(s1, g1, sem1)
                wait(s0, g0, sem0)
                scat(g0, d0)
                ld(2 * p + 2, d0, s0)
                start(s0, g0, sem0)
                wait(s1, g1, sem1)
                scat(g1, d1)
                return 0

            lax.fori_loop(0, nwt // 2 - 1, pair_body, 0)
            ld(nwt - 1, d1, s1)
            start(s1, g1, sem1)
            wait(s0, g0, sem0)
            scat(g0, d0)
            wait(s1, g1, sem1)
            scat(g1, d1)

        plsc.subcore_barrier()

        nf = R // FCH
        nf_iter = (nf + NS - 1) // NS

        def flush_body(i, _, out=out, tbase=tbase):
            chunk = i * NS + sid

            @pl.when(chunk < nf)
            def _():
                pltpu.sync_copy(
                    acc.at[pl.ds(chunk * FCH, FCH)],
                    out.at[cid, pl.ds(tbase + chunk * FCH, FCH)])
            return 0

        lax.fori_loop(0, nf_iter, flush_body, 0)
        plsc.subcore_barrier()


def _sc_scatter(hs, idx_pairs):
    mesh = plsc.VectorSubcoreMesh(core_axis_name="c", subcore_axis_name="s",
                                  num_cores=NC, num_subcores=NS)
    flat_idx = [a for pair in idx_pairs for a in pair]
    f = pl.kernel(
        _sc_body,
        out_type=[
            jax.ShapeDtypeStruct((NC, N0, C), jnp.float32),
            jax.ShapeDtypeStruct((NC, N1, C), jnp.float32),
            jax.ShapeDtypeStruct((NC, N2, C), jnp.float32),
        ],
        mesh=mesh,
        scratch_types=[
            pltpu.VMEM_SHARED((ACC_ROWS, C), jnp.float32),
            pltpu.VMEM((W,), jnp.int32),
            pltpu.VMEM((W,), jnp.int32),
            pltpu.VMEM((W,), jnp.int32),
            pltpu.VMEM((W,), jnp.int32),
            pltpu.VMEM((W, C), jnp.float32),
            pltpu.VMEM((W, C), jnp.float32),
            pltpu.VMEM((FCH, C), jnp.float32),
            pltpu.SemaphoreType.DMA,
            pltpu.SemaphoreType.DMA,
        ],
    )
    return f(*hs, *flat_idx)


@jax.jit
def kernel(x0, x1, x2, adj0_idx, adj1_idx, adj2_idx, inc1_idx, inc2_idx,
           W_same_0, W_same_1, W_same_2, W_h2l_0, W_h2l_1, W_l2h_1, W_l2h_2):
    h_s0, h_l2h1 = _matmuls(x0, [W_same_0, W_l2h_1])
    h_s1, h_h2l0, h_l2h2 = _matmuls(x1, [W_same_1, W_h2l_0, W_l2h_2])
    h_s2, h_h2l1 = _matmuls(x2, [W_same_2, W_h2l_1])

    idx_pairs = [
        _pad_idx(adj0_idx[0], adj0_idx[1], N0, N0),
        _pad_idx(inc1_idx[0], inc1_idx[1], N0, N1),
        _pad_idx(adj1_idx[0], adj1_idx[1], N1, N1),
        _pad_idx(inc2_idx[0], inc2_idx[1], N1, N2),
        _pad_idx(inc1_idx[1], inc1_idx[0], N1, N0),
        _pad_idx(adj2_idx[0], adj2_idx[1], N2, N2),
        _pad_idx(inc2_idx[1], inc2_idx[0], N2, N1),
    ]
    hs = (h_s0, h_h2l0, h_s1, h_h2l1, h_l2h1, h_s2, h_l2h2)
    p0, p1, p2 = _sc_scatter(hs, idx_pairs)
    return (_merge_sigmoid(p0), _merge_sigmoid(p1), _merge_sigmoid(p2))


# speedup vs baseline: 3.6263x; 1.2806x over previous
import jax
import jax.numpy as jnp
from jax import lax
from jax.experimental import pallas as pl
from jax.experimental.pallas import tpu as pltpu
from jax.experimental.pallas import tpu_sc as plsc

N0, N1, N2 = 10000, 30000, 20000
C = 128
NC, NS = 2, 16
NW = NC * NS
W = 128
K = 3
PADM = K * W * NW
R = 10000
DUMP = 8
ACC_ROWS = R + DUMP
FZ = 72
FF = 80


def _mm_body(x_ref, *refs):
    k = len(refs) // 2
    x = x_ref[...]
    for w_ref, o_ref in zip(refs[:k], refs[k:]):
        o_ref[...] = jnp.dot(x, w_ref[...], preferred_element_type=jnp.float32)


def _matmuls(x, ws, block=1000):
    n = x.shape[0]
    k = len(ws)
    return pl.pallas_call(
        _mm_body,
        grid=(n // block,),
        in_specs=[pl.BlockSpec((block, C), lambda i: (i, 0))]
        + [pl.BlockSpec((C, C), lambda i: (0, 0))] * k,
        out_specs=[pl.BlockSpec((block, C), lambda i: (i, 0))] * k,
        out_shape=[jax.ShapeDtypeStruct((n, C), jnp.float32)] * k,
    )(x, *ws)


def _merge_body(p_ref, o_ref):
    o_ref[...] = jax.nn.sigmoid(p_ref[0] + p_ref[1])


def _merge_sigmoid(p, block=1000):
    n = p.shape[1]
    return pl.pallas_call(
        _merge_body,
        grid=(n // block,),
        in_specs=[pl.BlockSpec((2, block, C), lambda i: (0, i, 0))],
        out_specs=pl.BlockSpec((block, C), lambda i: (i, 0)),
        out_shape=jax.ShapeDtypeStruct((n, C), jnp.float32),
    )(p)


def _pad_idx(dst, src, n_dst, n_src):
    nnz = dst.shape[0]
    m = (-nnz) % PADM
    dst = jnp.concatenate([dst, jnp.full((m,), n_dst, jnp.int32)])
    src = jnp.concatenate(
        [src, (jnp.arange(m, dtype=jnp.int32) * 997) % n_src])
    return dst, src


_TASKS = (
    (0, 0, [(0, 0), (1, 1)]),
    (1, 0, [(2, 2), (3, 3), (4, 4)]),
    (1, R, [(2, 2), (3, 3), (4, 4)]),
    (1, 2 * R, [(2, 2), (3, 3), (4, 4)]),
    (2, 0, [(5, 5), (6, 6)]),
    (2, R, [(5, 5), (6, 6)]),
)


def _sc_body(*args):
    hs = args[0:7]
    idxs = args[7:21]
    outs = args[21:24]
    ds_ = args[24:24 + K]
    ss_ = args[24 + K:24 + 2 * K]
    gs_ = args[24 + 2 * K:24 + 3 * K]
    acc = args[24 + 3 * K]
    gsems = args[24 + 3 * K + 1:24 + 4 * K + 1]
    ssems = args[24 + 4 * K + 1:24 + 5 * K + 1]
    fsem = args[24 + 5 * K + 1]

    cid = lax.axis_index("c")
    sid = lax.axis_index("s")
    wid = cid * NS + sid
    dump = jnp.full((16,), R, jnp.int32) + (sid % DUMP)
    zv = jnp.zeros((16,), jnp.float32)
    zb = gs_[0]

    for out_i, tbase, pass_list in _TASKS:
        out = outs[out_i]

        def zfill(r, _):
            for j in range(C // 16):
                zb[r, pl.ds(j * 16, 16)] = zv
            return 0

        lax.fori_loop(0, FZ, zfill, 0)
        nz = ACC_ROWS // FZ
        nz_iter = (nz + NS - 1) // NS

        def zero_body(i, _):
            chunk = i * NS + sid

            @pl.when(chunk < nz)
            def _():
                pltpu.async_copy(
                    zb.at[pl.ds(0, FZ)],
                    acc.at[pl.ds(chunk * FZ, FZ)], fsem)
            return 0

        def zero_drain(i, _):
            chunk = i * NS + sid

            @pl.when(chunk < nz)
            def _():
                pltpu.make_async_copy(
                    zb.at[pl.ds(0, FZ)],
                    acc.at[pl.ds(chunk * FZ, FZ)], fsem).wait()
            return 0

        lax.fori_loop(0, nz_iter, zero_body, 0)
        lax.fori_loop(0, nz_iter, zero_drain, 0)
        plsc.subcore_barrier()

        for h_i, idx_i in pass_list:
            h = hs[h_i]
            dst_hbm = idxs[2 * idx_i]
            src_hbm = idxs[2 * idx_i + 1]
            nwt = dst_hbm.shape[0] // (W * NW)
            w0 = wid * nwt

            def ld(i, b, dst_hbm=dst_hbm, src_hbm=src_hbm,
                   w0=w0, tbase=tbase):
                off = (w0 + i) * W
                d, s = ds_[b], ss_[b]
                pltpu.sync_copy(dst_hbm.at[pl.ds(off, W)], d)
                pltpu.sync_copy(src_hbm.at[pl.ds(off, W)], s)

                def tbody(j, _):
                    dd = d[pl.ds(j * 16, 16)]
                    l = dd - tbase
                    oor = (l < 0) | (l >= R)
                    d[pl.ds(j * 16, 16)] = jnp.where(oor, dump, l)
                    return 0

                lax.fori_loop(0, W // 16, tbody, 0)

            def start_g(b, h=h):
                pltpu.async_copy(h.at[ss_[b]], gs_[b], gsems[b])

            def wait_g(b, h=h):
                pltpu.make_async_copy(h.at[ss_[b]], gs_[b], gsems[b]).wait()

            def start_s(b):
                pltpu.async_copy(gs_[b], acc.at[ds_[b]], ssems[b], add=True)

            def wait_s(b):
                pltpu.make_async_copy(
                    gs_[b], acc.at[ds_[b]], ssems[b]).wait()

            for b in range(K):
                ld(b, b)
                start_g(b)

            def chunk_body(p, _, ld=ld, start_g=start_g, wait_g=wait_g,
                           start_s=start_s, wait_s=wait_s):
                for b in range(K):
                    wait_g(b)
                    start_s(b)
                for b in range(K):
                    wait_s(b)
                    ld(p * K + K + b, b)
                    start_g(b)
                return 0

            lax.fori_loop(0, nwt // K - 1, chunk_body, 0)
            for b in range(K):
                wait_g(b)
                start_s(b)
            for b in range(K):
                wait_s(b)

        plsc.subcore_barrier()

        nf = R // FF
        nf_iter = (nf + NS - 1) // NS

        def flush_body(i, _, out=out, tbase=tbase):
            chunk = i * NS + sid

            @pl.when(chunk < nf)
            def _():
                pltpu.async_copy(
                    acc.at[pl.ds(chunk * FF, FF)],
                    out.at[cid, pl.ds(tbase + chunk * FF, FF)], fsem)
            return 0

        def flush_drain(i, _, out=out, tbase=tbase):
            chunk = i * NS + sid

            @pl.when(chunk < nf)
            def _():
                pltpu.make_async_copy(
                    acc.at[pl.ds(chunk * FF, FF)],
                    out.at[cid, pl.ds(tbase + chunk * FF, FF)], fsem).wait()
            return 0

        lax.fori_loop(0, nf_iter, flush_body, 0)
        lax.fori_loop(0, nf_iter, flush_drain, 0)
        plsc.subcore_barrier()


def _sc_scatter(hs, idx_pairs):
    mesh = plsc.VectorSubcoreMesh(core_axis_name="c", subcore_axis_name="s",
                                  num_cores=NC, num_subcores=NS)
    flat_idx = [a for pair in idx_pairs for a in pair]
    f = pl.kernel(
        _sc_body,
        out_type=[
            jax.ShapeDtypeStruct((NC, N0, C), jnp.float32),
            jax.ShapeDtypeStruct((NC, N1, C), jnp.float32),
            jax.ShapeDtypeStruct((NC, N2, C), jnp.float32),
        ],
        mesh=mesh,
        scratch_types=(
            [pltpu.VMEM((W,), jnp.int32)] * K
            + [pltpu.VMEM((W,), jnp.int32)] * K
            + [pltpu.VMEM((W, C), jnp.float32)] * K
            + [pltpu.VMEM_SHARED((ACC_ROWS, C), jnp.float32)]
            + [pltpu.SemaphoreType.DMA] * (2 * K + 1)
        ),
    )
    return f(*hs, *flat_idx)


@jax.jit
def kernel(x0, x1, x2, adj0_idx, adj1_idx, adj2_idx, inc1_idx, inc2_idx,
           W_same_0, W_same_1, W_same_2, W_h2l_0, W_h2l_1, W_l2h_1, W_l2h_2):
    h_s0, h_l2h1 = _matmuls(x0, [W_same_0, W_l2h_1])
    h_s1, h_h2l0, h_l2h2 = _matmuls(x1, [W_same_1, W_h2l_0, W_l2h_2])
    h_s2, h_h2l1 = _matmuls(x2, [W_same_2, W_h2l_1])

    idx_pairs = [
        _pad_idx(adj0_idx[0], adj0_idx[1], N0, N0),
        _pad_idx(inc1_idx[0], inc1_idx[1], N0, N1),
        _pad_idx(adj1_idx[0], adj1_idx[1], N1, N1),
        _pad_idx(inc2_idx[0], inc2_idx[1], N1, N2),
        _pad_idx(inc1_idx[1], inc1_idx[0], N1, N0),
        _pad_idx(adj2_idx[0], adj2_idx[1], N2, N2),
        _pad_idx(inc2_idx[1], inc2_idx[0], N2, N1),
    ]
    hs = (h_s0, h_h2l0, h_s1, h_h2l1, h_l2h1, h_s2, h_l2h2)
    p0, p1, p2 = _sc_scatter(hs, idx_pairs)
    return (_merge_sigmoid(p0), _merge_sigmoid(p1), _merge_sigmoid(p2))
